# baseline (speedup 1.0000x reference)
import functools

import jax
import jax.numpy as jnp
from jax import lax
from jax.experimental import pallas as pl
from jax.experimental.pallas import tpu as pltpu
from jax.experimental.pallas import tpu_sc as plsc

ROWS = 128
COLS = 100001

KS0 = 0
KS1 = 42
KS2 = KS0 ^ KS1 ^ 0x1BD11BDA

_ROT_A = (13, 15, 26, 6)
_ROT_B = (17, 29, 16, 24)

SC_ROWS = 32
TC_ROWS = ROWS - SC_ROWS

LANES = 16
NUM_WORKERS = 32
SC_ROWS_PER_WORKER = SC_ROWS // NUM_WORKERS
SC_CHUNKS = (COLS + LANES - 1) // LANES

TC_RB = 8
TC_W = 512
TC_UNROLL = 9


def _rotl(x, r):
    return (x << jnp.uint32(r)) | (x >> jnp.uint32(32 - r))


def _threefry_word(i_arr):
    ks = (jnp.uint32(KS0), jnp.uint32(KS1), jnp.uint32(KS2))
    x1 = i_arr + ks[1]
    x0 = x1
    x1 = _rotl(x1, _ROT_A[0]) ^ x0
    for r in _ROT_A[1:]:
        x0 = x0 + x1
        x1 = _rotl(x1, r) ^ x0
    x0 = x0 + ks[1]
    x1 = x1 + ks[2] + jnp.uint32(1)
    rots = (_ROT_A, _ROT_B)
    for i in range(1, 5):
        for r in rots[i % 2]:
            x0 = x0 + x1
            x1 = _rotl(x1, r) ^ x0
        x0 = x0 + ks[(i + 1) % 3]
        x1 = x1 + ks[(i + 2) % 3] + jnp.uint32(i + 1)
    return x0 ^ x1


def _build_sc(base_row_offset, rows_per_worker):
    mesh = plsc.VectorSubcoreMesh(core_axis_name="c", subcore_axis_name="s")

    @functools.partial(
        pl.kernel,
        mesh=mesh,
        out_type=jax.ShapeDtypeStruct((NUM_WORKERS, LANES), jnp.float32),
        scratch_types=[pltpu.VMEM((LANES,), jnp.float32)],
    )
    def k(out_hbm, out_v):
        wid = lax.axis_index("s") * 2 + lax.axis_index("c")
        base_row = base_row_offset + wid * rows_per_worker
        lane = lax.iota(jnp.uint32, LANES)
        lane_i = lax.iota(jnp.int32, LANES)
        rowlane = [
            (base_row + r).astype(jnp.uint32) * jnp.uint32(COLS) + lane
            for r in range(rows_per_worker)
        ]
        neg1 = jnp.full((LANES,), -1, jnp.int32)

        def body(t, carry):
            bvs, bis = carry
            off = t.astype(jnp.uint32) * jnp.uint32(LANES)
            new_bvs, new_bis = [], []
            for r in range(rows_per_worker):
                word = _threefry_word(rowlane[r] + off)
                v = (word >> jnp.uint32(9)).astype(jnp.int32)
                better = v > bvs[r]
                new_bvs.append(jnp.where(better, v, bvs[r]))
                new_bis.append(jnp.where(better, t, bis[r]))
            return tuple(new_bvs), tuple(new_bis)

        init = (
            tuple(neg1 for _ in range(rows_per_worker)),
            tuple(jnp.zeros((LANES,), jnp.int32) for _ in range(rows_per_worker)),
        )
        bvs, bis = lax.fori_loop(0, SC_CHUNKS - 1, body, init)
        bvs, bis = list(bvs), list(bis)

        t_last = SC_CHUNKS - 1
        valid = (lane_i + t_last * LANES) < COLS
        for r in range(rows_per_worker):
            word = _threefry_word(rowlane[r] + jnp.uint32(t_last * LANES))
            v = (word >> jnp.uint32(9)).astype(jnp.int32)
            better = jnp.logical_and(v > bvs[r], valid)
            bvs[r] = jnp.where(better, v, bvs[r])
            bis[r] = jnp.where(better, jnp.int32(t_last), bis[r])

        res = []
        for r in range(rows_per_worker):
            cols = bis[r] * LANES + lane_i
            m = jnp.int32(-1)
            idx = jnp.int32(0x7FFFFFFF)
            for l in range(LANES):
                v = bvs[r][l]
                i = cols[l]
                better = jnp.logical_or(v > m, jnp.logical_and(v == m, i < idx))
                m = jnp.where(better, v, m)
                idx = jnp.where(better, i, idx)
            res.append(idx.astype(jnp.float32))

        outv = jnp.zeros((LANES,), jnp.float32)
        for r in range(rows_per_worker):
            outv = jnp.where(lane_i == r, res[r], outv)
        out_v[...] = outv
        pltpu.sync_copy(out_v, out_hbm.at[wid])

    return k


def _tc_body(out_ref, *, n_cols, row_block, width, unroll):
    p = pl.program_id(0)
    r0 = p * row_block
    row_ids = jax.lax.broadcasted_iota(jnp.uint32, (row_block, width), 0) + jnp.uint32(r0)
    col_iota = jax.lax.broadcasted_iota(jnp.uint32, (row_block, width), 1)
    base = row_ids * jnp.uint32(n_cols) + col_iota
    neg1 = jnp.full((row_block, width), -1, jnp.int32)
    group = width * unroll
    n_groups = (n_cols + group - 1) // group

    def step_unmasked(t, carry):
        bv, bi = carry
        g0 = t.astype(jnp.uint32) * jnp.uint32(group)
        words = [
            _threefry_word(base + (g0 + jnp.uint32(u * width))) for u in range(unroll)
        ]
        ct0 = t * unroll
        for u in range(unroll):
            v = (words[u] >> jnp.uint32(9)).astype(jnp.int32)
            better = v > bv
            bv = jnp.where(better, v, bv)
            bi = jnp.where(better, ct0 + u, bi)
        return bv, bi

    bv, bi = lax.fori_loop(
        0, n_groups - 1, step_unmasked, (neg1, jnp.zeros((row_block, width), jnp.int32))
    )

    t_last = n_groups - 1
    g0 = jnp.uint32(t_last * group)
    for u in range(unroll):
        c = col_iota + (g0 + jnp.uint32(u * width))
        word = _threefry_word(base + (g0 + jnp.uint32(u * width)))
        v = (word >> jnp.uint32(9)).astype(jnp.int32)
        better = jnp.logical_and(v > bv, c <= jnp.uint32(n_cols - 1))
        bv = jnp.where(better, v, bv)
        bi = jnp.where(better, t_last * unroll + u, bi)

    col = bi * width + col_iota.astype(jnp.int32)
    m = jnp.max(bv, axis=1, keepdims=True)
    cand = jnp.where(bv == m, col, jnp.full((row_block, width), 0x7FFFFFFF, jnp.int32))
    idx = jnp.min(cand, axis=1, keepdims=True)
    out_ref[...] = jnp.broadcast_to(idx.astype(jnp.float32), (row_block, 128))


def _build_tc(n_rows, n_cols=COLS, row_block=TC_RB, width=TC_W, unroll=TC_UNROLL):
    grid = n_rows // row_block
    return pl.pallas_call(
        functools.partial(
            _tc_body, n_cols=n_cols, row_block=row_block, width=width, unroll=unroll
        ),
        out_shape=jax.ShapeDtypeStruct((n_rows, 128), jnp.float32),
        grid=(grid,),
        out_specs=pl.BlockSpec((row_block, 128), lambda p: (p, 0)),
    )


_sc_kernel_cache = []


def _get_sc_kernel():
    if not _sc_kernel_cache:
        _sc_kernel_cache.append(_build_sc(TC_ROWS, SC_ROWS_PER_WORKER))
    return _sc_kernel_cache[0]


_tc_kernel = _build_tc(TC_ROWS)


def kernel(x):
    del x
    sc_out = _get_sc_kernel()()[:, :SC_ROWS_PER_WORKER].reshape(SC_ROWS)
    tc_out = _tc_kernel()[:, 0]
    return jnp.concatenate([tc_out, sc_out])


# speedup vs baseline: 1.0019x; 1.0019x over previous
import functools

import jax
import jax.numpy as jnp
from jax import lax
from jax.experimental import pallas as pl
from jax.experimental.pallas import tpu as pltpu
from jax.experimental.pallas import tpu_sc as plsc

ROWS = 128
COLS = 100001

KS0 = 0
KS1 = 42
KS2 = KS0 ^ KS1 ^ 0x1BD11BDA

_ROT_A = (13, 15, 26, 6)
_ROT_B = (17, 29, 16, 24)

SC_ROWS = 32
TC_ROWS = ROWS - SC_ROWS

LANES = 16
NUM_WORKERS = 32
SC_ROWS_PER_WORKER = SC_ROWS // NUM_WORKERS
SC_CHUNKS = (COLS + LANES - 1) // LANES

TC_RB = 8
TC_W = 512
TC_UNROLL = 8


def _rotl(x, r):
    return (x << jnp.uint32(r)) | (x >> jnp.uint32(32 - r))


def _threefry_word(i_arr):
    ks = (jnp.uint32(KS0), jnp.uint32(KS1), jnp.uint32(KS2))
    x1 = i_arr + ks[1]
    x0 = x1
    x1 = _rotl(x1, _ROT_A[0]) ^ x0
    for r in _ROT_A[1:]:
        x0 = x0 + x1
        x1 = _rotl(x1, r) ^ x0
    x0 = x0 + ks[1]
    x1 = x1 + ks[2] + jnp.uint32(1)
    rots = (_ROT_A, _ROT_B)
    for i in range(1, 5):
        for r in rots[i % 2]:
            x0 = x0 + x1
            x1 = _rotl(x1, r) ^ x0
        x0 = x0 + ks[(i + 1) % 3]
        x1 = x1 + ks[(i + 2) % 3] + jnp.uint32(i + 1)
    return x0 ^ x1


def _build_sc(base_row_offset, rows_per_worker):
    mesh = plsc.VectorSubcoreMesh(core_axis_name="c", subcore_axis_name="s")

    @functools.partial(
        pl.kernel,
        mesh=mesh,
        out_type=jax.ShapeDtypeStruct((NUM_WORKERS, LANES), jnp.float32),
        scratch_types=[pltpu.VMEM((LANES,), jnp.float32)],
    )
    def k(out_hbm, out_v):
        wid = lax.axis_index("s") * 2 + lax.axis_index("c")
        base_row = base_row_offset + wid * rows_per_worker
        lane = lax.iota(jnp.uint32, LANES)
        lane_i = lax.iota(jnp.int32, LANES)
        rowlane = [
            (base_row + r).astype(jnp.uint32) * jnp.uint32(COLS) + lane
            for r in range(rows_per_worker)
        ]
        neg1 = jnp.full((LANES,), -1, jnp.int32)

        def body(t, carry):
            bvs, bis = carry
            off = t.astype(jnp.uint32) * jnp.uint32(LANES)
            new_bvs, new_bis = [], []
            for r in range(rows_per_worker):
                word = _threefry_word(rowlane[r] + off)
                v = (word >> jnp.uint32(9)).astype(jnp.int32)
                better = v > bvs[r]
                new_bvs.append(jnp.where(better, v, bvs[r]))
                new_bis.append(jnp.where(better, t, bis[r]))
            return tuple(new_bvs), tuple(new_bis)

        init = (
            tuple(neg1 for _ in range(rows_per_worker)),
            tuple(jnp.zeros((LANES,), jnp.int32) for _ in range(rows_per_worker)),
        )
        bvs, bis = lax.fori_loop(0, SC_CHUNKS - 1, body, init)
        bvs, bis = list(bvs), list(bis)

        t_last = SC_CHUNKS - 1
        valid = (lane_i + t_last * LANES) < COLS
        for r in range(rows_per_worker):
            word = _threefry_word(rowlane[r] + jnp.uint32(t_last * LANES))
            v = (word >> jnp.uint32(9)).astype(jnp.int32)
            better = jnp.logical_and(v > bvs[r], valid)
            bvs[r] = jnp.where(better, v, bvs[r])
            bis[r] = jnp.where(better, jnp.int32(t_last), bis[r])

        res = []
        for r in range(rows_per_worker):
            cols = bis[r] * LANES + lane_i
            m = jnp.int32(-1)
            idx = jnp.int32(0x7FFFFFFF)
            for l in range(LANES):
                v = bvs[r][l]
                i = cols[l]
                better = jnp.logical_or(v > m, jnp.logical_and(v == m, i < idx))
                m = jnp.where(better, v, m)
                idx = jnp.where(better, i, idx)
            res.append(idx.astype(jnp.float32))

        outv = jnp.zeros((LANES,), jnp.float32)
        for r in range(rows_per_worker):
            outv = jnp.where(lane_i == r, res[r], outv)
        out_v[...] = outv
        pltpu.sync_copy(out_v, out_hbm.at[wid])

    return k


def _tc_body(out_ref, *, n_cols, row_block, width, unroll):
    p = pl.program_id(0)
    r0 = p * row_block
    row_ids = jax.lax.broadcasted_iota(jnp.uint32, (row_block, width), 0) + jnp.uint32(r0)
    col_iota = jax.lax.broadcasted_iota(jnp.uint32, (row_block, width), 1)
    base = row_ids * jnp.uint32(n_cols) + col_iota
    neg1 = jnp.full((row_block, width), -1, jnp.int32)
    group = width * unroll
    n_groups = (n_cols + group - 1) // group

    def step_unmasked(t, carry):
        bv, bi = carry
        g0 = t.astype(jnp.uint32) * jnp.uint32(group)
        words = [
            _threefry_word(base + (g0 + jnp.uint32(u * width))) for u in range(unroll)
        ]
        ct0 = t * unroll
        for u in range(unroll):
            v = (words[u] >> jnp.uint32(9)).astype(jnp.int32)
            better = v > bv
            bv = jnp.where(better, v, bv)
            bi = jnp.where(better, ct0 + u, bi)
        return bv, bi

    bv, bi = lax.fori_loop(
        0, n_groups - 1, step_unmasked, (neg1, jnp.zeros((row_block, width), jnp.int32))
    )

    t_last = n_groups - 1
    g0 = jnp.uint32(t_last * group)
    for u in range(unroll):
        c = col_iota + (g0 + jnp.uint32(u * width))
        word = _threefry_word(base + (g0 + jnp.uint32(u * width)))
        v = (word >> jnp.uint32(9)).astype(jnp.int32)
        better = jnp.logical_and(v > bv, c <= jnp.uint32(n_cols - 1))
        bv = jnp.where(better, v, bv)
        bi = jnp.where(better, t_last * unroll + u, bi)

    col = bi * width + col_iota.astype(jnp.int32)
    m = jnp.max(bv, axis=1, keepdims=True)
    cand = jnp.where(bv == m, col, jnp.full((row_block, width), 0x7FFFFFFF, jnp.int32))
    idx = jnp.min(cand, axis=1, keepdims=True)
    out_ref[...] = jnp.broadcast_to(idx.astype(jnp.float32), (row_block, 128))


def _build_tc(n_rows, n_cols=COLS, row_block=TC_RB, width=TC_W, unroll=TC_UNROLL):
    grid = n_rows // row_block
    return pl.pallas_call(
        functools.partial(
            _tc_body, n_cols=n_cols, row_block=row_block, width=width, unroll=unroll
        ),
        out_shape=jax.ShapeDtypeStruct((n_rows, 128), jnp.float32),
        grid=(grid,),
        out_specs=pl.BlockSpec((row_block, 128), lambda p: (p, 0)),
    )


_sc_kernel_cache = []


def _get_sc_kernel():
    if not _sc_kernel_cache:
        _sc_kernel_cache.append(_build_sc(TC_ROWS, SC_ROWS_PER_WORKER))
    return _sc_kernel_cache[0]


_tc_kernel = _build_tc(TC_ROWS)


def kernel(x):
    del x
    sc_out = _get_sc_kernel()()[:, :SC_ROWS_PER_WORKER].reshape(SC_ROWS)
    tc_out = _tc_kernel()[:, 0]
    return jnp.concatenate([tc_out, sc_out])
